# baseline (speedup 1.0000x reference)
import functools

import jax
import jax.numpy as jnp
from jax import lax
from jax.experimental import pallas as pl
from jax.experimental.pallas import tpu as pltpu
from jax.experimental.pallas import tpu_sc as plsc

ROWS = 128
COLS = 32768
K = 8

TC_COLS = 22528
SC_COLS = COLS - TC_COLS


_NET8 = [
    (0, 1), (2, 3), (4, 5), (6, 7),
    (0, 2), (1, 3), (4, 6), (5, 7),
    (1, 2), (5, 6),
    (0, 4), (1, 5), (2, 6), (3, 7),
    (2, 4), (3, 5),
    (1, 2), (3, 4), (5, 6),
]

_BITONIC8 = [
    (0, 4), (1, 5), (2, 6), (3, 7),
    (0, 2), (1, 3), (4, 6), (5, 7),
    (0, 1), (2, 3), (4, 5), (6, 7),
]


def _apply_net(vs, net):
    vs = list(vs)
    for i, j in net:
        a, b = vs[i], vs[j]
        vs[i] = jnp.maximum(a, b)
        vs[j] = jnp.minimum(a, b)
    return vs


def _merge_top8(avs, bvs):
    c = [jnp.maximum(avs[i], bvs[K - 1 - i]) for i in range(K)]
    return _apply_net(c, _BITONIC8)


def _halve(vs):
    w = vs[0].shape[1] // 2
    a = [v[:, :w] for v in vs]
    b = [v[:, w:] for v in vs]
    return _merge_top8(a, b)


NC = 2
NS = 16
NW = NC * NS
RPW = ROWS // NW
SLANES = 16
SGRP = K * SLANES
SNGRP = SC_COLS // SGRP

_mesh = plsc.VectorSubcoreMesh(core_axis_name="c", subcore_axis_name="s")


@functools.partial(
    pl.kernel,
    mesh=_mesh,
    out_type=jax.ShapeDtypeStruct((ROWS, K * SLANES), jnp.float32),
    scratch_types=[
        pltpu.VMEM((SC_COLS,), jnp.float32),
        pltpu.VMEM((SC_COLS,), jnp.float32),
        pltpu.VMEM((RPW, K * SLANES), jnp.float32),
        pltpu.SemaphoreType.DMA,
        pltpu.SemaphoreType.DMA,
    ],
)
def _sc_topk(x_hbm, out_hbm, buf0, buf1, res, sem0, sem1):
    wid = lax.axis_index("s") * NC + lax.axis_index("c")
    row0 = wid * RPW
    bufs = (buf0, buf1)
    sems = (sem0, sem1)

    def dma(r, b):
        return pltpu.make_async_copy(
            x_hbm.at[row0 + r, pl.ds(TC_COLS, SC_COLS)], bufs[b], sems[b])

    dma(0, 0).start()
    for r in range(RPW):
        if r + 1 < RPW:
            dma(r + 1, (r + 1) % 2).start()
        buf = bufs[r % 2]
        dma(r, r % 2).wait()

        neg = jnp.full((SLANES,), -jnp.inf, jnp.float32)

        def body(g, accs):
            base = g * SGRP
            vs = [buf[pl.ds(base + k * SLANES, SLANES)] for k in range(K)]
            vs = _apply_net(vs, _NET8)
            return tuple(_merge_top8(list(accs), vs))

        accs = lax.fori_loop(0, SNGRP, body, (neg,) * K)
        for j in range(K):
            res[r, pl.ds(j * SLANES, SLANES)] = accs[j]

    pltpu.sync_copy(res, out_hbm.at[pl.ds(row0, RPW)])


RBLK = 8
NRB = ROWS // RBLK
TLANES = 128
TCHUNK = K * TLANES
TNCHUNK = TC_COLS // TCHUNK
NCHAINS = 4


def _tc_kernel(x_ref, o_ref, acc_ref):
    step = pl.program_id(0)
    accs = [None] * NCHAINS
    for c in range(TNCHUNK):
        base = c * TCHUNK
        vs = [x_ref[:, base + k * TLANES:base + (k + 1) * TLANES]
              for k in range(K)]
        vs = _apply_net(vs, _NET8)
        ch = c % NCHAINS
        accs[ch] = vs if accs[ch] is None else _merge_top8(accs[ch], vs)
    acc = _merge_top8(_merge_top8(accs[0], accs[1]),
                      _merge_top8(accs[2], accs[3]))
    acc_ref[pl.ds(step * RBLK, RBLK), :] = jnp.concatenate(acc, axis=1)

    @pl.when(step == NRB - 1)
    def _finalize():
        fin = [acc_ref[:, k * TLANES:(k + 1) * TLANES] for k in range(K)]
        while fin[0].shape[1] > 1:
            fin = _halve(fin)
        o_ref[...] = jnp.concatenate(fin, axis=1)


def _tc_topk(x):
    return pl.pallas_call(
        _tc_kernel,
        grid=(NRB,),
        in_specs=[pl.BlockSpec((RBLK, TC_COLS), lambda i: (i, 0))],
        out_specs=pl.BlockSpec((ROWS, K), lambda i: (0, 0)),
        out_shape=jax.ShapeDtypeStruct((ROWS, K), jnp.float32),
        scratch_shapes=[pltpu.VMEM((ROWS, K * TLANES), jnp.float32)],
    )(x)


def _merge_kernel(c_ref, t_ref, o_ref):
    fin = [c_ref[:, j * SLANES:(j + 1) * SLANES] for j in range(K)]
    while fin[0].shape[1] > 1:
        fin = _halve(fin)
    tc = [t_ref[:, j:j + 1] for j in range(K)]
    out = _merge_top8(tc, fin)
    o_ref[...] = jnp.concatenate(out, axis=1)


def _merge(cand_sc, tc8):
    return pl.pallas_call(
        _merge_kernel,
        in_specs=[pl.BlockSpec((ROWS, K * SLANES), lambda: (0, 0)),
                  pl.BlockSpec((ROWS, K), lambda: (0, 0))],
        out_specs=pl.BlockSpec((ROWS, K), lambda: (0, 0)),
        out_shape=jax.ShapeDtypeStruct((ROWS, K), jnp.float32),
    )(cand_sc, tc8)


def kernel(input):
    tc8 = _tc_topk(input)
    cand_sc = _sc_topk(input)
    return _merge(cand_sc, tc8)


# speedup vs baseline: 1.9830x; 1.9830x over previous
import jax
import jax.numpy as jnp
from jax.experimental import pallas as pl
from jax.experimental.pallas import tpu as pltpu

ROWS = 128
COLS = 32768
K = 8
RBLK = 8
NRB = ROWS // RBLK
LANES = 128
CHUNK = K * LANES
NCHUNK = COLS // CHUNK

_NET8 = [
    (0, 1), (2, 3), (4, 5), (6, 7),
    (0, 2), (1, 3), (4, 6), (5, 7),
    (1, 2), (5, 6),
    (0, 4), (1, 5), (2, 6), (3, 7),
    (2, 4), (3, 5),
    (1, 2), (3, 4), (5, 6),
]

_BITONIC8 = [
    (0, 4), (1, 5), (2, 6), (3, 7),
    (0, 2), (1, 3), (4, 6), (5, 7),
    (0, 1), (2, 3), (4, 5), (6, 7),
]


def _apply_net(vs, net):
    vs = list(vs)
    for i, j in net:
        a, b = vs[i], vs[j]
        vs[i] = jnp.maximum(a, b)
        vs[j] = jnp.minimum(a, b)
    return vs


def _merge_top8(avs, bvs):
    c = [jnp.maximum(avs[i], bvs[K - 1 - i]) for i in range(K)]
    return _apply_net(c, _BITONIC8)


NCHAINS = 4
NSTREAM = 4
CPS = NCHUNK // NSTREAM


def _topk_kernel(*refs):
    x_refs = refs[:NSTREAM]
    o_ref = refs[NSTREAM]
    acc_ref = refs[NSTREAM + 1]
    step = pl.program_id(0)
    accs = [None] * NCHAINS
    for c in range(NCHUNK):
        x_ref = x_refs[c // CPS]
        base = (c % CPS) * CHUNK
        vs = [x_ref[:, base + k * LANES:base + (k + 1) * LANES]
              for k in range(K)]
        vs = _apply_net(vs, _NET8)
        ch = c % NCHAINS
        accs[ch] = vs if accs[ch] is None else _merge_top8(accs[ch], vs)
    acc = _merge_top8(_merge_top8(accs[0], accs[1]),
                      _merge_top8(accs[2], accs[3]))
    acc_ref[pl.ds(step * RBLK, RBLK), :] = jnp.concatenate(acc, axis=1)

    @pl.when(step == NRB - 1)
    def _finalize():
        fin = [acc_ref[:, k * LANES:(k + 1) * LANES] for k in range(K)]
        w = LANES
        while w > 1:
            w //= 2
            a = [v[:, :w] for v in fin]
            b = [v[:, w:] for v in fin]
            fin = _merge_top8(a, b)
        o_ref[...] = jnp.concatenate(fin, axis=1)


def kernel(input):
    return pl.pallas_call(
        _topk_kernel,
        grid=(NRB,),
        in_specs=[pl.BlockSpec((RBLK, COLS // NSTREAM),
                               lambda i, s=s: (i, s))
                  for s in range(NSTREAM)],
        out_specs=pl.BlockSpec((ROWS, K), lambda i: (0, 0)),
        out_shape=jax.ShapeDtypeStruct((ROWS, K), jnp.float32),
        scratch_shapes=[pltpu.VMEM((ROWS, K * LANES), jnp.float32)],
    )(*([input] * NSTREAM))


# speedup vs baseline: 2.5060x; 1.2637x over previous
import jax
import jax.numpy as jnp
from jax.experimental import pallas as pl
from jax.experimental.pallas import tpu as pltpu

ROWS = 128
COLS = 32768
K = 8
RBLK = 16
NRB = ROWS // RBLK
LANES = 128
CHUNK = K * LANES
NCHUNK = COLS // CHUNK

_NET8 = [
    (0, 1), (2, 3), (4, 5), (6, 7),
    (0, 2), (1, 3), (4, 6), (5, 7),
    (1, 2), (5, 6),
    (0, 4), (1, 5), (2, 6), (3, 7),
    (2, 4), (3, 5),
    (1, 2), (3, 4), (5, 6),
]

_BITONIC8 = [
    (0, 4), (1, 5), (2, 6), (3, 7),
    (0, 2), (1, 3), (4, 6), (5, 7),
    (0, 1), (2, 3), (4, 5), (6, 7),
]


def _apply_net(vs, net):
    vs = list(vs)
    for i, j in net:
        a, b = vs[i], vs[j]
        vs[i] = jnp.maximum(a, b)
        vs[j] = jnp.minimum(a, b)
    return vs


def _merge_top8(avs, bvs):
    c = [jnp.maximum(avs[i], bvs[K - 1 - i]) for i in range(K)]
    return _apply_net(c, _BITONIC8)


NCHAINS = 4
NSTREAM = 4
CPS = NCHUNK // NSTREAM


def _topk_kernel(*refs):
    x_refs = refs[:NSTREAM]
    o_ref = refs[NSTREAM]
    acc_ref = refs[NSTREAM + 1]
    step = pl.program_id(0)
    accs = [None] * NCHAINS
    for c in range(NCHUNK):
        x_ref = x_refs[c // CPS]
        base = (c % CPS) * CHUNK
        vs = [x_ref[:, base + k * LANES:base + (k + 1) * LANES]
              for k in range(K)]
        vs = _apply_net(vs, _NET8)
        ch = c % NCHAINS
        accs[ch] = vs if accs[ch] is None else _merge_top8(accs[ch], vs)
    acc = _merge_top8(_merge_top8(accs[0], accs[1]),
                      _merge_top8(accs[2], accs[3]))
    acc_ref[pl.ds(step * RBLK, RBLK), :] = jnp.concatenate(acc, axis=1)

    @pl.when(step == NRB - 1)
    def _finalize():
        fin = [acc_ref[:, k * LANES:(k + 1) * LANES] for k in range(K)]
        w = LANES
        while w > 1:
            w //= 2
            a = [v[:, :w] for v in fin]
            b = [v[:, w:] for v in fin]
            fin = _merge_top8(a, b)
        o_ref[...] = jnp.concatenate(fin, axis=1)


def kernel(input):
    return pl.pallas_call(
        _topk_kernel,
        grid=(NRB,),
        in_specs=[pl.BlockSpec((RBLK, COLS // NSTREAM),
                               lambda i, s=s: (i, s))
                  for s in range(NSTREAM)],
        out_specs=pl.BlockSpec((ROWS, K), lambda i: (0, 0)),
        out_shape=jax.ShapeDtypeStruct((ROWS, K), jnp.float32),
        scratch_shapes=[pltpu.VMEM((ROWS, K * LANES), jnp.float32)],
    )(*([input] * NSTREAM))


# speedup vs baseline: 2.8668x; 1.1440x over previous
import jax
import jax.numpy as jnp
from jax.experimental import pallas as pl
from jax.experimental.pallas import tpu as pltpu

ROWS = 128
COLS = 32768
K = 8
RBLK = 32
NRB = ROWS // RBLK
LANES = 128
CHUNK = K * LANES
NCHUNK = COLS // CHUNK

_NET8 = [
    (0, 1), (2, 3), (4, 5), (6, 7),
    (0, 2), (1, 3), (4, 6), (5, 7),
    (1, 2), (5, 6),
    (0, 4), (1, 5), (2, 6), (3, 7),
    (2, 4), (3, 5),
    (1, 2), (3, 4), (5, 6),
]

_BITONIC8 = [
    (0, 4), (1, 5), (2, 6), (3, 7),
    (0, 2), (1, 3), (4, 6), (5, 7),
    (0, 1), (2, 3), (4, 5), (6, 7),
]


def _apply_net(vs, net):
    vs = list(vs)
    for i, j in net:
        a, b = vs[i], vs[j]
        vs[i] = jnp.maximum(a, b)
        vs[j] = jnp.minimum(a, b)
    return vs


def _merge_top8(avs, bvs):
    c = [jnp.maximum(avs[i], bvs[K - 1 - i]) for i in range(K)]
    return _apply_net(c, _BITONIC8)


NCHAINS = 4
NSTREAM = 4
CPS = NCHUNK // NSTREAM


def _topk_kernel(*refs):
    x_refs = refs[:NSTREAM]
    o_ref = refs[NSTREAM]
    acc_ref = refs[NSTREAM + 1]
    step = pl.program_id(0)
    accs = [None] * NCHAINS
    for c in range(NCHUNK):
        x_ref = x_refs[c // CPS]
        base = (c % CPS) * CHUNK
        vs = [x_ref[:, base + k * LANES:base + (k + 1) * LANES]
              for k in range(K)]
        vs = _apply_net(vs, _NET8)
        ch = c % NCHAINS
        accs[ch] = vs if accs[ch] is None else _merge_top8(accs[ch], vs)
    acc = _merge_top8(_merge_top8(accs[0], accs[1]),
                      _merge_top8(accs[2], accs[3]))
    acc_ref[pl.ds(step * RBLK, RBLK), :] = jnp.concatenate(acc, axis=1)

    @pl.when(step == NRB - 1)
    def _finalize():
        fin = [acc_ref[:, k * LANES:(k + 1) * LANES] for k in range(K)]
        w = LANES
        while w > 1:
            w //= 2
            a = [v[:, :w] for v in fin]
            b = [v[:, w:] for v in fin]
            fin = _merge_top8(a, b)
        o_ref[...] = jnp.concatenate(fin, axis=1)


def kernel(input):
    return pl.pallas_call(
        _topk_kernel,
        grid=(NRB,),
        in_specs=[pl.BlockSpec((RBLK, COLS // NSTREAM),
                               lambda i, s=s: (i, s))
                  for s in range(NSTREAM)],
        out_specs=pl.BlockSpec((ROWS, K), lambda i: (0, 0)),
        out_shape=jax.ShapeDtypeStruct((ROWS, K), jnp.float32),
        scratch_shapes=[pltpu.VMEM((ROWS, K * LANES), jnp.float32)],
    )(*([input] * NSTREAM))
